# baseline (speedup 1.0000x reference)
import jax
import jax.numpy as jnp
from jax.experimental import pallas as pl
from jax.experimental.pallas import tpu as pltpu

_TIME = 4096
_BATCH = 1024
_NOUT = 2
_GAP = 8
_ROWS = _TIME * 16
_CROWS = 4096
_NCHUNK = _ROWS // _CROWS
_NBUF = 4


def _compute(o, t):
    d = o - t
    sq = d * d
    part0 = jnp.sum(sq.reshape(_CROWS // 8, 8, 128), axis=0)
    o4 = o.reshape(_CROWS // 128, 8, 16, 128)
    rowsum = jnp.sum(o4, axis=1)
    tobs = t.reshape(_CROWS // 128, 8, 16, 128)[:, 0, :, :]
    d1 = rowsum - 8.0 * tobs
    sq1 = d1 * d1
    part1 = jnp.sum(sq1, axis=0)
    return part0, part1


def _loss_kernel(o_hbm, t_hbm, out_ref, obuf, tbuf, acc0_ref, acc1_ref, sems):
    def start_in(c, b):
        pltpu.make_async_copy(
            o_hbm.at[pl.ds(c * _CROWS, _CROWS), :], obuf.at[b], sems.at[b, 0]
        ).start()
        pltpu.make_async_copy(
            t_hbm.at[pl.ds(c * _CROWS, _CROWS), :], tbuf.at[b], sems.at[b, 1]
        ).start()

    def wait_in(c, b):
        pltpu.make_async_copy(
            o_hbm.at[pl.ds(c * _CROWS, _CROWS), :], obuf.at[b], sems.at[b, 0]
        ).wait()
        pltpu.make_async_copy(
            t_hbm.at[pl.ds(c * _CROWS, _CROWS), :], tbuf.at[b], sems.at[b, 1]
        ).wait()

    acc0_ref[...] = jnp.zeros((8, 128), jnp.float32)
    acc1_ref[...] = jnp.zeros((16, 128), jnp.float32)

    for b in range(_NBUF):
        start_in(b, b)

    def body(i, _):
        b = jax.lax.rem(i, _NBUF)
        wait_in(i, b)
        part0, part1 = _compute(obuf[b], tbuf[b])
        acc0_ref[...] += part0
        acc1_ref[...] += part1

        @pl.when(i + _NBUF < _NCHUNK)
        def _next():
            start_in(i + _NBUF, b)

        return 0

    jax.lax.fori_loop(0, _NCHUNK, body, 0)

    row0 = jax.lax.broadcasted_iota(jnp.int32, (8, 128), 0)
    s0 = jnp.sum(jnp.where(row0 % 2 == 0, acc0_ref[...], 0.0))
    row1 = jax.lax.broadcasted_iota(jnp.int32, (16, 128), 0)
    s1 = jnp.sum(jnp.where(row1 % 2 == 1, acc1_ref[...], 0.0))
    n0 = float(_TIME * _BATCH)
    n1 = float((_TIME // _GAP) * _BATCH)
    out_ref[0, 0] = 0.5 * (s0 / n0) + 0.5 * (s1 / (64.0 * n1))


def _rowview(x):
    return (
        x.reshape(_TIME, 8, 128, _NOUT)
        .transpose(0, 1, 3, 2)
        .reshape(_ROWS, 128)
    )


def kernel(output, target):
    o2 = _rowview(output)
    t2 = _rowview(target)
    out = pl.pallas_call(
        _loss_kernel,
        in_specs=[
            pl.BlockSpec(memory_space=pl.ANY),
            pl.BlockSpec(memory_space=pl.ANY),
        ],
        out_specs=pl.BlockSpec(memory_space=pltpu.SMEM),
        out_shape=jax.ShapeDtypeStruct((1, 1), jnp.float32),
        scratch_shapes=[
            pltpu.VMEM((_NBUF, _CROWS, 128), jnp.float32),
            pltpu.VMEM((_NBUF, _CROWS, 128), jnp.float32),
            pltpu.VMEM((8, 128), jnp.float32),
            pltpu.VMEM((16, 128), jnp.float32),
            pltpu.SemaphoreType.DMA((_NBUF, 2)),
        ],
    )(o2, t2)
    return out[0, 0]


# speedup vs baseline: 1.0355x; 1.0355x over previous
import jax
import jax.numpy as jnp
from jax.experimental import pallas as pl
from jax.experimental.pallas import tpu as pltpu

_TIME = 4096
_BATCH = 1024
_NOUT = 2
_GAP = 8
_ROWS = _TIME * 16
_CROWS = 2048
_NCHUNK = _ROWS // _CROWS
_NBUF = 3


def _compute(o, t):
    d = o - t
    sq = d * d
    part0 = jnp.sum(sq.reshape(_CROWS // 8, 8, 128), axis=0)
    o4 = o.reshape(_CROWS // 128, 8, 16, 128)
    rowsum = jnp.sum(o4, axis=1)
    tobs = t.reshape(_CROWS // 128, 8, 16, 128)[:, 0, :, :]
    d1 = rowsum - 8.0 * tobs
    sq1 = d1 * d1
    part1 = jnp.sum(sq1, axis=0)
    return part0, part1


def _loss_kernel(o_hbm, t_hbm, out_ref, obuf, tbuf, acc0_ref, acc1_ref, sems):
    def start_in(c, b):
        pltpu.make_async_copy(
            o_hbm.at[pl.ds(c * _CROWS, _CROWS), :], obuf.at[b], sems.at[b, 0]
        ).start()
        pltpu.make_async_copy(
            t_hbm.at[pl.ds(c * _CROWS, _CROWS), :], tbuf.at[b], sems.at[b, 1]
        ).start()

    def wait_in(c, b):
        pltpu.make_async_copy(
            o_hbm.at[pl.ds(c * _CROWS, _CROWS), :], obuf.at[b], sems.at[b, 0]
        ).wait()
        pltpu.make_async_copy(
            t_hbm.at[pl.ds(c * _CROWS, _CROWS), :], tbuf.at[b], sems.at[b, 1]
        ).wait()

    acc0_ref[...] = jnp.zeros((8, 128), jnp.float32)
    acc1_ref[...] = jnp.zeros((16, 128), jnp.float32)

    for b in range(_NBUF):
        start_in(b, b)

    def body(i, _):
        b = jax.lax.rem(i, _NBUF)
        wait_in(i, b)
        part0, part1 = _compute(obuf[b], tbuf[b])
        acc0_ref[...] += part0
        acc1_ref[...] += part1

        @pl.when(i + _NBUF < _NCHUNK)
        def _next():
            start_in(i + _NBUF, b)

        return 0

    jax.lax.fori_loop(0, _NCHUNK, body, 0)

    row0 = jax.lax.broadcasted_iota(jnp.int32, (8, 128), 0)
    s0 = jnp.sum(jnp.where(row0 % 2 == 0, acc0_ref[...], 0.0))
    row1 = jax.lax.broadcasted_iota(jnp.int32, (16, 128), 0)
    s1 = jnp.sum(jnp.where(row1 % 2 == 1, acc1_ref[...], 0.0))
    n0 = float(_TIME * _BATCH)
    n1 = float((_TIME // _GAP) * _BATCH)
    out_ref[0, 0] = 0.5 * (s0 / n0) + 0.5 * (s1 / (64.0 * n1))


def _rowview(x):
    return (
        x.reshape(_TIME, 8, 128, _NOUT)
        .transpose(0, 1, 3, 2)
        .reshape(_ROWS, 128)
    )


def kernel(output, target):
    o2 = _rowview(output)
    t2 = _rowview(target)
    out = pl.pallas_call(
        _loss_kernel,
        in_specs=[
            pl.BlockSpec(memory_space=pl.ANY),
            pl.BlockSpec(memory_space=pl.ANY),
        ],
        out_specs=pl.BlockSpec(memory_space=pltpu.SMEM),
        out_shape=jax.ShapeDtypeStruct((1, 1), jnp.float32),
        scratch_shapes=[
            pltpu.VMEM((_NBUF, _CROWS, 128), jnp.float32),
            pltpu.VMEM((_NBUF, _CROWS, 128), jnp.float32),
            pltpu.VMEM((8, 128), jnp.float32),
            pltpu.VMEM((16, 128), jnp.float32),
            pltpu.SemaphoreType.DMA((_NBUF, 2)),
        ],
    )(o2, t2)
    return out[0, 0]


# speedup vs baseline: 1.0642x; 1.0277x over previous
import jax
import jax.numpy as jnp
from jax.experimental import pallas as pl
from jax.experimental.pallas import tpu as pltpu

_TIME = 4096
_BATCH = 1024
_NOUT = 2
_GAP = 8
_ROWS = _TIME * 16
_CROWS = 2048
_NCHUNK = _ROWS // _CROWS
_NBUF = 4


def _compute(o, t):
    d = o - t
    sq = d * d
    part0 = jnp.sum(sq.reshape(_CROWS // 8, 8, 128), axis=0)
    o4 = o.reshape(_CROWS // 128, 8, 16, 128)
    rowsum = jnp.sum(o4, axis=1)
    tobs = t.reshape(_CROWS // 128, 8, 16, 128)[:, 0, :, :]
    d1 = rowsum - 8.0 * tobs
    sq1 = d1 * d1
    part1 = jnp.sum(sq1, axis=0)
    return part0, part1


def _loss_kernel(o_hbm, t_hbm, out_ref, obuf, tbuf, acc0_ref, acc1_ref, sems):
    def start_in(c, b):
        pltpu.make_async_copy(
            o_hbm.at[pl.ds(c * _CROWS, _CROWS), :], obuf.at[b], sems.at[b, 0]
        ).start()
        pltpu.make_async_copy(
            t_hbm.at[pl.ds(c * _CROWS, _CROWS), :], tbuf.at[b], sems.at[b, 1]
        ).start()

    def wait_in(c, b):
        pltpu.make_async_copy(
            o_hbm.at[pl.ds(c * _CROWS, _CROWS), :], obuf.at[b], sems.at[b, 0]
        ).wait()
        pltpu.make_async_copy(
            t_hbm.at[pl.ds(c * _CROWS, _CROWS), :], tbuf.at[b], sems.at[b, 1]
        ).wait()

    acc0_ref[...] = jnp.zeros((8, 128), jnp.float32)
    acc1_ref[...] = jnp.zeros((16, 128), jnp.float32)

    for b in range(_NBUF):
        start_in(b, b)

    def body(i, _):
        b = jax.lax.rem(i, _NBUF)
        wait_in(i, b)
        part0, part1 = _compute(obuf[b], tbuf[b])
        acc0_ref[...] += part0
        acc1_ref[...] += part1

        @pl.when(i + _NBUF < _NCHUNK)
        def _next():
            start_in(i + _NBUF, b)

        return 0

    jax.lax.fori_loop(0, _NCHUNK, body, 0)

    row0 = jax.lax.broadcasted_iota(jnp.int32, (8, 128), 0)
    s0 = jnp.sum(jnp.where(row0 % 2 == 0, acc0_ref[...], 0.0))
    row1 = jax.lax.broadcasted_iota(jnp.int32, (16, 128), 0)
    s1 = jnp.sum(jnp.where(row1 % 2 == 1, acc1_ref[...], 0.0))
    n0 = float(_TIME * _BATCH)
    n1 = float((_TIME // _GAP) * _BATCH)
    out_ref[0, 0] = 0.5 * (s0 / n0) + 0.5 * (s1 / (64.0 * n1))


def _rowview(x):
    return (
        x.reshape(_TIME, 8, 128, _NOUT)
        .transpose(0, 1, 3, 2)
        .reshape(_ROWS, 128)
    )


def kernel(output, target):
    o2 = _rowview(output)
    t2 = _rowview(target)
    out = pl.pallas_call(
        _loss_kernel,
        in_specs=[
            pl.BlockSpec(memory_space=pl.ANY),
            pl.BlockSpec(memory_space=pl.ANY),
        ],
        out_specs=pl.BlockSpec(memory_space=pltpu.SMEM),
        out_shape=jax.ShapeDtypeStruct((1, 1), jnp.float32),
        scratch_shapes=[
            pltpu.VMEM((_NBUF, _CROWS, 128), jnp.float32),
            pltpu.VMEM((_NBUF, _CROWS, 128), jnp.float32),
            pltpu.VMEM((8, 128), jnp.float32),
            pltpu.VMEM((16, 128), jnp.float32),
            pltpu.SemaphoreType.DMA((_NBUF, 2)),
        ],
    )(o2, t2)
    return out[0, 0]


# speedup vs baseline: 1.0752x; 1.0103x over previous
import jax
import jax.numpy as jnp
from jax.experimental import pallas as pl
from jax.experimental.pallas import tpu as pltpu

_TIME = 4096
_BATCH = 1024
_NOUT = 2
_GAP = 8
_ROWS = _TIME * 16
_CROWS = 1024
_NCHUNK = _ROWS // _CROWS
_NBUF = 6


def _compute(o, t):
    d = o - t
    sq = d * d
    part0 = jnp.sum(sq.reshape(_CROWS // 8, 8, 128), axis=0)
    o4 = o.reshape(_CROWS // 128, 8, 16, 128)
    rowsum = jnp.sum(o4, axis=1)
    tobs = t.reshape(_CROWS // 128, 8, 16, 128)[:, 0, :, :]
    d1 = rowsum - 8.0 * tobs
    sq1 = d1 * d1
    part1 = jnp.sum(sq1, axis=0)
    return part0, part1


def _loss_kernel(o_hbm, t_hbm, out_ref, obuf, tbuf, acc0_ref, acc1_ref, sems):
    def start_in(c, b):
        pltpu.make_async_copy(
            o_hbm.at[pl.ds(c * _CROWS, _CROWS), :], obuf.at[b], sems.at[b, 0]
        ).start()
        pltpu.make_async_copy(
            t_hbm.at[pl.ds(c * _CROWS, _CROWS), :], tbuf.at[b], sems.at[b, 1]
        ).start()

    def wait_in(c, b):
        pltpu.make_async_copy(
            o_hbm.at[pl.ds(c * _CROWS, _CROWS), :], obuf.at[b], sems.at[b, 0]
        ).wait()
        pltpu.make_async_copy(
            t_hbm.at[pl.ds(c * _CROWS, _CROWS), :], tbuf.at[b], sems.at[b, 1]
        ).wait()

    acc0_ref[...] = jnp.zeros((8, 128), jnp.float32)
    acc1_ref[...] = jnp.zeros((16, 128), jnp.float32)

    for b in range(_NBUF):
        start_in(b, b)

    def body(i, _):
        b = jax.lax.rem(i, _NBUF)
        wait_in(i, b)
        part0, part1 = _compute(obuf[b], tbuf[b])
        acc0_ref[...] += part0
        acc1_ref[...] += part1

        @pl.when(i + _NBUF < _NCHUNK)
        def _next():
            start_in(i + _NBUF, b)

        return 0

    jax.lax.fori_loop(0, _NCHUNK, body, 0)

    row0 = jax.lax.broadcasted_iota(jnp.int32, (8, 128), 0)
    s0 = jnp.sum(jnp.where(row0 % 2 == 0, acc0_ref[...], 0.0))
    row1 = jax.lax.broadcasted_iota(jnp.int32, (16, 128), 0)
    s1 = jnp.sum(jnp.where(row1 % 2 == 1, acc1_ref[...], 0.0))
    n0 = float(_TIME * _BATCH)
    n1 = float((_TIME // _GAP) * _BATCH)
    out_ref[0, 0] = 0.5 * (s0 / n0) + 0.5 * (s1 / (64.0 * n1))


def _rowview(x):
    return (
        x.reshape(_TIME, 8, 128, _NOUT)
        .transpose(0, 1, 3, 2)
        .reshape(_ROWS, 128)
    )


def kernel(output, target):
    o2 = _rowview(output)
    t2 = _rowview(target)
    out = pl.pallas_call(
        _loss_kernel,
        in_specs=[
            pl.BlockSpec(memory_space=pl.ANY),
            pl.BlockSpec(memory_space=pl.ANY),
        ],
        out_specs=pl.BlockSpec(memory_space=pltpu.SMEM),
        out_shape=jax.ShapeDtypeStruct((1, 1), jnp.float32),
        scratch_shapes=[
            pltpu.VMEM((_NBUF, _CROWS, 128), jnp.float32),
            pltpu.VMEM((_NBUF, _CROWS, 128), jnp.float32),
            pltpu.VMEM((8, 128), jnp.float32),
            pltpu.VMEM((16, 128), jnp.float32),
            pltpu.SemaphoreType.DMA((_NBUF, 2)),
        ],
    )(o2, t2)
    return out[0, 0]
